# baseline (speedup 1.0000x reference)
import functools

import jax
import jax.numpy as jnp
from jax import lax
from jax.experimental import pallas as pl
from jax.experimental.pallas import tpu as pltpu
from jax.experimental.pallas import tpu_sc as plsc

DIM = 768
NUM_TABLES = 6
CHUNK = 64
NBUF = 2


@functools.lru_cache(maxsize=None)
def _make_gather(B: int, D: int):
    info = plsc.get_sparse_core_info()
    NC, NS = info.num_cores, info.num_subcores
    NW = NC * NS
    assert B % (8 * NW) == 0
    b_per_w = B // NW
    assert b_per_w % CHUNK == 0
    n_chunks = b_per_w // CHUNK
    n_steps = NUM_TABLES * n_chunks

    mesh = plsc.VectorSubcoreMesh(core_axis_name="c", subcore_axis_name="s")

    @functools.partial(
        pl.kernel,
        mesh=mesh,
        out_type=[jax.ShapeDtypeStruct((B, D), jnp.float32)] * (2 * NUM_TABLES),
        scratch_types=(
            [pltpu.VMEM((n_chunks, CHUNK), jnp.int32)]
            + [pltpu.VMEM((CHUNK, D), jnp.float32)] * NBUF
            + [pltpu.SemaphoreType.DMA] * (2 * NBUF + 1)
        ),
    )
    def gather6(idx_hbm, t0, t1, t2, t3, t4, t5, *rest):
        outs = rest[:2 * NUM_TABLES]
        idx_v = rest[2 * NUM_TABLES]
        rows = rest[2 * NUM_TABLES + 1:2 * NUM_TABLES + 1 + NBUF]
        sems = rest[2 * NUM_TABLES + 1 + NBUF:]
        gsem = sems[:NBUF]
        ssem = sems[NBUF:2 * NBUF]
        dsem = sems[2 * NBUF]
        tables = (t0, t1, t2, t3, t4, t5)
        wid = lax.axis_index("s") * NC + lax.axis_index("c")
        base = wid * b_per_w
        pltpu.sync_copy(
            idx_hbm.at[pl.ds(wid * n_chunks, n_chunks)], idx_v)

        def start_gather(s):
            t, c = divmod(s, n_chunks)
            b = s % NBUF
            return pltpu.async_copy(
                tables[t].at[idx_v.at[c]], rows[b], gsem[b])

        def start_scatter(s):
            t, c = divmod(s, n_chunks)
            b = s % NBUF
            dst = pl.ds(base + c * CHUNK, CHUNK)
            return pltpu.async_copy(rows[b], outs[t].at[dst], ssem[b])

        def start_dup(s):
            t, c = divmod(s, n_chunks)
            dst = pl.ds(base + c * CHUNK, CHUNK)
            return pltpu.async_copy(outs[t].at[dst], outs[11 - t].at[dst], dsem)

        g_h = [None] * n_steps
        s_h = [None] * n_steps
        d_h = [None] * n_steps
        for s in range(NBUF):
            g_h[s] = start_gather(s)
        for s in range(n_steps):
            g_h[s].wait()
            s_h[s] = start_scatter(s)
            nxt = s + NBUF
            if nxt < n_steps:
                s_h[s].wait()
                d_h[s] = start_dup(s)
                g_h[nxt] = start_gather(nxt)
        for s in range(n_steps - NBUF, n_steps):
            s_h[s].wait()
            d_h[s] = start_dup(s)
        for h in d_h:
            h.wait()

    return gather6


def kernel(inputs, W0, W1, W2, W3, W4, W5):
    batch, seq = inputs.shape
    flat_idx = inputs.reshape(-1, CHUNK).astype(jnp.int32)
    outs = _make_gather(batch * seq, DIM)(flat_idx, W0, W1, W2, W3, W4, W5)
    return tuple(o.reshape(batch, seq, DIM) for o in outs)


# speedup vs baseline: 17.8166x; 17.8166x over previous
import functools

import jax
import jax.numpy as jnp
from jax import lax
from jax.experimental import pallas as pl
from jax.experimental.pallas import tpu as pltpu
from jax.experimental.pallas import tpu_sc as plsc

DIM = 768
CHUNK = 64
NBUF = 2


@functools.lru_cache(maxsize=None)
def _make_gather(B: int, D: int, dests: tuple):
    n_tab = len(dests)
    n_out = max(max(d) for d in dests) + 1
    info = plsc.get_sparse_core_info()
    NC, NS = info.num_cores, info.num_subcores
    NW = NC * NS
    assert B % (8 * NW) == 0
    b_per_w = B // NW
    assert b_per_w % CHUNK == 0
    n_chunks = b_per_w // CHUNK
    n_steps = n_tab * n_chunks

    mesh = plsc.VectorSubcoreMesh(core_axis_name="c", subcore_axis_name="s")

    @functools.partial(
        pl.kernel,
        mesh=mesh,
        out_type=[jax.ShapeDtypeStruct((B, D), jnp.float32)] * n_out,
        scratch_types=(
            [pltpu.VMEM((n_chunks, CHUNK), jnp.int32)]
            + [pltpu.VMEM((CHUNK, D), jnp.float32)] * NBUF
            + [pltpu.SemaphoreType.DMA] * (2 * NBUF)
        ),
    )
    def gather_call(idx_hbm, *rest):
        tables = rest[:n_tab]
        outs = rest[n_tab:n_tab + n_out]
        idx_v = rest[n_tab + n_out]
        rows = rest[n_tab + n_out + 1:n_tab + n_out + 1 + NBUF]
        sems = rest[n_tab + n_out + 1 + NBUF:]
        gsem = sems[:NBUF]
        ssem = sems[NBUF:]
        wid = lax.axis_index("s") * NC + lax.axis_index("c")
        base = wid * b_per_w
        pltpu.sync_copy(
            idx_hbm.at[pl.ds(wid * n_chunks, n_chunks)], idx_v)

        def start_gather(s):
            t, c = divmod(s, n_chunks)
            b = s % NBUF
            return pltpu.async_copy(
                tables[t].at[idx_v.at[c]], rows[b], gsem[b])

        def start_scatters(s):
            t, c = divmod(s, n_chunks)
            b = s % NBUF
            dst = pl.ds(base + c * CHUNK, CHUNK)
            return tuple(
                pltpu.async_copy(rows[b], outs[o].at[dst], ssem[b])
                for o in dests[t])

        g_h = [None] * n_steps
        s_h = [None] * n_steps
        for s in range(NBUF):
            g_h[s] = start_gather(s)
        for s in range(n_steps):
            g_h[s].wait()
            s_h[s] = start_scatters(s)
            nxt = s + NBUF
            if nxt < n_steps:
                for h in s_h[s]:
                    h.wait()
                g_h[nxt] = start_gather(nxt)
        for s in range(n_steps - NBUF, n_steps):
            for h in s_h[s]:
                h.wait()

    return gather_call


def kernel(inputs, W0, W1, W2, W3, W4, W5):
    batch, seq = inputs.shape
    B = batch * seq
    flat_idx = inputs.reshape(-1, CHUNK).astype(jnp.int32)
    v0, v1 = _make_gather(B, DIM, ((0,), (1,)))(flat_idx, W0, W1)
    v2, v3, v4, v5, d6, d7 = _make_gather(
        B, DIM, ((0,), (1,), (2, 5), (3, 4)))(flat_idx, W2, W3, W4, W5)
    outs = (v0, v1, v2, v3, v4, v5, d6, d7, v3, v2, v1, v0)
    return tuple(o.reshape(batch, seq, DIM) for o in outs)


# speedup vs baseline: 20.2822x; 1.1384x over previous
import functools

import jax
import jax.numpy as jnp
from jax import lax
from jax.experimental import pallas as pl
from jax.experimental.pallas import tpu as pltpu
from jax.experimental.pallas import tpu_sc as plsc

DIM = 768
CHUNK = 64
NBUF = 2


@functools.lru_cache(maxsize=None)
def _make_gather(B: int, D: int, dests: tuple):
    n_tab = len(dests)
    n_out = max(max(d) for d in dests) + 1
    info = plsc.get_sparse_core_info()
    NC, NS = info.num_cores, info.num_subcores
    NW = NC * NS
    assert B % (8 * NW) == 0
    b_per_w = B // NW
    assert b_per_w % CHUNK == 0
    n_chunks = b_per_w // CHUNK
    n_steps = n_tab * n_chunks

    mesh = plsc.VectorSubcoreMesh(core_axis_name="c", subcore_axis_name="s")

    @functools.partial(
        pl.kernel,
        mesh=mesh,
        out_type=[jax.ShapeDtypeStruct((B, D), jnp.float32)] * n_out,
        scratch_types=(
            [pltpu.VMEM((n_chunks, CHUNK), jnp.int32)]
            + [pltpu.VMEM((CHUNK, D), jnp.float32)] * NBUF
            + [pltpu.SemaphoreType.DMA] * (2 * NBUF)
        ),
    )
    def gather_call(idx_hbm, *rest):
        tables = rest[:n_tab]
        outs = rest[n_tab:n_tab + n_out]
        idx_v = rest[n_tab + n_out]
        rows = rest[n_tab + n_out + 1:n_tab + n_out + 1 + NBUF]
        sems = rest[n_tab + n_out + 1 + NBUF:]
        gsem = sems[:NBUF]
        ssem = sems[NBUF:]
        wid = lax.axis_index("s") * NC + lax.axis_index("c")
        base = wid * b_per_w
        pltpu.sync_copy(
            idx_hbm.at[pl.ds(wid * n_chunks, n_chunks)], idx_v)

        def start_gather(s):
            t, c = divmod(s, n_chunks)
            b = s % NBUF
            return pltpu.async_copy(
                tables[t].at[idx_v.at[c]], rows[b], gsem[b])

        def start_scatters(s):
            t, c = divmod(s, n_chunks)
            b = s % NBUF
            dst = pl.ds(base + c * CHUNK, CHUNK)
            return tuple(
                pltpu.async_copy(rows[b], outs[o].at[dst], ssem[b])
                for o in dests[t])

        g_h = [None] * n_steps
        s_h = [None] * n_steps
        for s in range(NBUF):
            g_h[s] = start_gather(s)
        for s in range(n_steps):
            g_h[s].wait()
            s_h[s] = start_scatters(s)
            nxt = s + NBUF
            if nxt < n_steps:
                for h in s_h[s]:
                    h.wait()
                g_h[nxt] = start_gather(nxt)
        for s in range(n_steps - NBUF, n_steps):
            for h in s_h[s]:
                h.wait()

    return gather_call


@functools.lru_cache(maxsize=None)
def _make_tc_copy(B: int, D: int, n: int, bs: int = 256):
    def body(*refs):
        for i, o in zip(refs[:n], refs[n:]):
            o[...] = i[...]

    return pl.pallas_call(
        body,
        grid=(B // bs,),
        in_specs=[pl.BlockSpec((bs, D), lambda i: (i, 0))] * n,
        out_specs=[pl.BlockSpec((bs, D), lambda i: (i, 0))] * n,
        out_shape=[jax.ShapeDtypeStruct((B, D), jnp.float32)] * n,
    )


def kernel(inputs, W0, W1, W2, W3, W4, W5):
    batch, seq = inputs.shape
    B = batch * seq
    flat_idx = inputs.reshape(-1, CHUNK).astype(jnp.int32)
    v0, v1, v2 = _make_gather(B, DIM, ((0,), (1,), (2,)))(
        flat_idx, W0, W1, W2)
    v3, v4, v5, d6, d7, d8 = _make_gather(
        B, DIM, ((0, 5), (1, 4), (2, 3)))(flat_idx, W3, W4, W5)
    d9, d10, d11 = _make_tc_copy(B, DIM, 3)(v2, v1, v0)
    outs = (v0, v1, v2, v3, v4, v5, d6, d7, d8, d9, d10, d11)
    return tuple(o.reshape(batch, seq, DIM) for o in outs)
